# baseline (device time: 18698 ns/iter reference)
import jax
import jax.numpy as jnp
from jax import lax
from jax.experimental import pallas as pl
from jax.experimental.pallas import tpu as pltpu

N_DEV = 4
CAP = 400
HDR = 2
UNIT = 16
NBITS = 5


def _chunks(cr):
    out = []
    for b in range(NBITS - 1, -1, -1):
        hi = (cr >> (b + 1)) << (b + 1)
        out.append((b, hi * UNIT, UNIT << b, ((cr >> b) & 1) != 0))
    return out


def _a2a_body(x_ref, dest_row_ref, hdr_ref, out_ref,
              send_ref, stg_ref, hdr_stg_ref,
              dsend_sems, drecv_sems, hsend_sems, hrecv_sems):
    me = lax.axis_index("i")
    n, d = x_ref.shape
    f32, bf16, i32 = jnp.float32, jnp.bfloat16, jnp.int32

    stg_ref[...] = jnp.zeros((N_DEV, CAP, d), bf16)

    bar = pltpu.get_barrier_semaphore()
    for k in range(1, N_DEV):
        pl.semaphore_signal(
            bar, inc=1,
            device_id=((me + k) % N_DEV,),
            device_id_type=pl.DeviceIdType.MESH,
        )
    pl.semaphore_wait(bar, N_DEV - 1)

    hdr_stg_ref[me] = hdr_ref[...]
    hdr_sends = []
    for k in range(1, N_DEV):
        rdma = pltpu.make_async_remote_copy(
            src_ref=hdr_ref,
            dst_ref=hdr_stg_ref.at[me],
            send_sem=hsend_sems.at[k],
            recv_sem=hrecv_sems.at[k],
            device_id=((me + k) % N_DEV,),
            device_id_type=pl.DeviceIdType.MESH,
        )
        rdma.start()
        hdr_sends.append(rdma)

    dest_row = dest_row_ref[...]
    iota4 = lax.broadcasted_iota(i32, (N_DEV, n), 0)
    A = (iota4 == dest_row.astype(i32)).astype(f32)
    cnt_out = [jnp.sum(A[r]) for r in range(N_DEV)]

    def cnt_out_at(dst):
        return jnp.where(
            dst == 0, cnt_out[0],
            jnp.where(dst == 1, cnt_out[1],
                      jnp.where(dst == 2, cnt_out[2], cnt_out[3])),
        )
    c = A
    sh = 1
    while sh < n:
        c = c + jnp.concatenate(
            [jnp.zeros((N_DEV, sh), f32), c[:, : n - sh]], axis=1
        )
        sh *= 2
    rank_row = jnp.sum(A * (c - A), axis=0, keepdims=True)
    f_int = (dest_row * CAP + rank_row).astype(i32)
    xb = x_ref[...].astype(bf16)
    iota_p = lax.broadcasted_iota(i32, (CAP, n), 0)

    def cr_of(cnt_scalar):
        return (cnt_scalar.astype(i32) + UNIT - 1) >> 4

    send_chunks = []
    for k in (2, 1, 3):
        dst = (me + k) % N_DEV
        P = (iota_p == f_int - dst * CAP).astype(bf16)
        send_ref[dst] = lax.dot_general(
            P, xb, (((1,), (0,)), ((), ())), preferred_element_type=f32
        ).astype(bf16)
        cr = cr_of(cnt_out_at(dst))
        for b, off, sz, active in _chunks(cr):
            rdma = pltpu.make_async_remote_copy(
                src_ref=send_ref.at[dst, pl.ds(off, sz)],
                dst_ref=stg_ref.at[me, pl.ds(off, sz)],
                send_sem=dsend_sems.at[k, b],
                recv_sem=drecv_sems.at[k, b],
                device_id=(dst,),
                device_id_type=pl.DeviceIdType.MESH,
            )
            pl.when(active)(rdma.start)
            send_chunks.append((active, rdma))

    P_me = (iota_p == f_int - me * CAP).astype(bf16)
    own_bf = lax.dot_general(
        P_me, xb, (((1,), (0,)), ((), ())), preferred_element_type=f32
    ).astype(bf16)

    for k in range(1, N_DEV):
        recv = pltpu.make_async_remote_copy(
            src_ref=hdr_ref,
            dst_ref=hdr_stg_ref.at[(me - k) % N_DEV],
            send_sem=hsend_sems.at[k],
            recv_sem=hrecv_sems.at[k],
            device_id=((me - k) % N_DEV,),
            device_id_type=pl.DeviceIdType.MESH,
        )
        recv.wait_recv()

    me_f = me.astype(f32)
    cnt = [
        jnp.sum((hdr_stg_ref[s] == me_f).astype(f32)) for s in range(N_DEV)
    ]
    o1 = cnt[0]
    o2 = o1 + cnt[1]
    o3 = o2 + cnt[2]
    k_col = lax.broadcasted_iota(i32, (n, 1), 0).astype(f32)
    s_col = (
        (k_col >= o1).astype(f32)
        + (k_col >= o2).astype(f32)
        + (k_col >= o3).astype(f32)
    )
    start_col = jnp.where(
        s_col == 0.0, 0.0,
        jnp.where(s_col == 1.0, o1, jnp.where(s_col == 2.0, o2, o3)),
    )
    j_int = (k_col - start_col).astype(i32)
    s_int = s_col.astype(i32)
    iota_c = lax.broadcasted_iota(i32, (n, CAP), 1)

    def q_mat(src):
        return ((iota_c == j_int) & (s_int == src)).astype(bf16)

    def cnt_at(src):
        return jnp.where(
            src == 0, cnt[0],
            jnp.where(src == 1, cnt[1], jnp.where(src == 2, cnt[2], cnt[3])),
        )

    acc = lax.dot_general(
        q_mat(me), own_bf, (((1,), (0,)), ((), ())),
        preferred_element_type=f32,
    )
    for k in (1, 3, 2):
        src = (me - k) % N_DEV
        cr = cr_of(cnt_at(src))
        for b, off, sz, active in _chunks(cr):
            recv = pltpu.make_async_remote_copy(
                src_ref=send_ref.at[src, pl.ds(off, sz)],
                dst_ref=stg_ref.at[src, pl.ds(off, sz)],
                send_sem=dsend_sems.at[k, b],
                recv_sem=drecv_sems.at[k, b],
                device_id=(src,),
                device_id_type=pl.DeviceIdType.MESH,
            )
            pl.when(active)(recv.wait_recv)
        acc = acc + lax.dot_general(
            q_mat(src), stg_ref[src],
            (((1,), (0,)), ((), ())), preferred_element_type=f32,
        )
    out_ref[...] = acc

    for rdma in hdr_sends:
        rdma.wait_send()
    for active, rdma in send_chunks:
        pl.when(active)(rdma.wait_send)


def kernel(x, dest):
    n, d = x.shape
    dest_f = dest.astype(jnp.float32)
    dest_row = dest_f.reshape(1, n)
    hdr = dest_f.reshape(HDR, d)

    return pl.pallas_call(
        _a2a_body,
        out_shape=jax.ShapeDtypeStruct((n, d), jnp.float32),
        in_specs=[
            pl.BlockSpec(memory_space=pltpu.VMEM),
            pl.BlockSpec(memory_space=pltpu.VMEM),
            pl.BlockSpec(memory_space=pltpu.VMEM),
        ],
        out_specs=pl.BlockSpec(memory_space=pltpu.VMEM),
        scratch_shapes=[
            pltpu.VMEM((N_DEV, CAP, d), jnp.bfloat16),
            pltpu.VMEM((N_DEV, CAP, d), jnp.bfloat16),
            pltpu.VMEM((N_DEV, HDR, d), jnp.float32),
            pltpu.SemaphoreType.DMA((N_DEV, NBITS)),
            pltpu.SemaphoreType.DMA((N_DEV, NBITS)),
            pltpu.SemaphoreType.DMA((N_DEV,)),
            pltpu.SemaphoreType.DMA((N_DEV,)),
        ],
        compiler_params=pltpu.CompilerParams(collective_id=0),
    )(x, dest_row, hdr)


# device time: 17278 ns/iter; 1.0822x vs baseline; 1.0822x over previous
import jax
import jax.numpy as jnp
from jax import lax
from jax.experimental import pallas as pl
from jax.experimental.pallas import tpu as pltpu

N_DEV = 4
CAP = 288
HDR = 2
UNIT = 16
NBITS = 5


def _chunks(cr):
    out = []
    for b in range(NBITS - 1, -1, -1):
        hi = (cr >> (b + 1)) << (b + 1)
        out.append((b, hi * UNIT, UNIT << b, ((cr >> b) & 1) != 0))
    return out


def _a2a_body(x_ref, dest_row_ref, hdr_ref, out_ref,
              send_ref, stg_ref, hdr_stg_ref,
              dsend_sems, drecv_sems, hsend_sems, hrecv_sems):
    me = lax.axis_index("i")
    n, d = x_ref.shape
    f32, bf16, i32 = jnp.float32, jnp.bfloat16, jnp.int32

    stg_ref[...] = jnp.zeros((N_DEV, CAP, d), bf16)

    bar = pltpu.get_barrier_semaphore()
    for k in range(1, N_DEV):
        pl.semaphore_signal(
            bar, inc=1,
            device_id=((me + k) % N_DEV,),
            device_id_type=pl.DeviceIdType.MESH,
        )
    pl.semaphore_wait(bar, N_DEV - 1)

    hdr_stg_ref[me] = hdr_ref[...]
    hdr_sends = []
    for k in range(1, N_DEV):
        rdma = pltpu.make_async_remote_copy(
            src_ref=hdr_ref,
            dst_ref=hdr_stg_ref.at[me],
            send_sem=hsend_sems.at[k],
            recv_sem=hrecv_sems.at[k],
            device_id=((me + k) % N_DEV,),
            device_id_type=pl.DeviceIdType.MESH,
        )
        rdma.start()
        hdr_sends.append(rdma)

    dest_row = dest_row_ref[...]
    iota4 = lax.broadcasted_iota(i32, (N_DEV, n), 0)
    A = (iota4 == dest_row.astype(i32)).astype(f32)
    cnt_out = [jnp.sum(A[r]) for r in range(N_DEV)]

    def cnt_out_at(dst):
        return jnp.where(
            dst == 0, cnt_out[0],
            jnp.where(dst == 1, cnt_out[1],
                      jnp.where(dst == 2, cnt_out[2], cnt_out[3])),
        )
    c = A
    sh = 1
    while sh < n:
        c = c + jnp.concatenate(
            [jnp.zeros((N_DEV, sh), f32), c[:, : n - sh]], axis=1
        )
        sh *= 2
    rank_row = jnp.sum(A * (c - A), axis=0, keepdims=True)
    f_int = (dest_row * CAP + rank_row).astype(i32)
    xb = x_ref[...].astype(bf16)
    iota_p = lax.broadcasted_iota(i32, (CAP, n), 0)

    def cr_of(cnt_scalar):
        return (cnt_scalar.astype(i32) + UNIT - 1) >> 4

    send_chunks = []
    for k in (2, 1, 3):
        dst = (me + k) % N_DEV
        P = (iota_p == f_int - dst * CAP).astype(bf16)
        send_ref[dst] = lax.dot_general(
            P, xb, (((1,), (0,)), ((), ())), preferred_element_type=f32
        ).astype(bf16)
        cr = cr_of(cnt_out_at(dst))
        for b, off, sz, active in _chunks(cr):
            rdma = pltpu.make_async_remote_copy(
                src_ref=send_ref.at[dst, pl.ds(off, sz)],
                dst_ref=stg_ref.at[me, pl.ds(off, sz)],
                send_sem=dsend_sems.at[k, b],
                recv_sem=drecv_sems.at[k, b],
                device_id=(dst,),
                device_id_type=pl.DeviceIdType.MESH,
            )
            pl.when(active)(rdma.start)
            send_chunks.append((active, rdma))

    P_me = (iota_p == f_int - me * CAP).astype(bf16)
    own_bf = lax.dot_general(
        P_me, xb, (((1,), (0,)), ((), ())), preferred_element_type=f32
    ).astype(bf16)

    for k in range(1, N_DEV):
        recv = pltpu.make_async_remote_copy(
            src_ref=hdr_ref,
            dst_ref=hdr_stg_ref.at[(me - k) % N_DEV],
            send_sem=hsend_sems.at[k],
            recv_sem=hrecv_sems.at[k],
            device_id=((me - k) % N_DEV,),
            device_id_type=pl.DeviceIdType.MESH,
        )
        recv.wait_recv()

    me_f = me.astype(f32)
    cnt = [
        jnp.sum((hdr_stg_ref[s] == me_f).astype(f32)) for s in range(N_DEV)
    ]
    o1 = cnt[0]
    o2 = o1 + cnt[1]
    o3 = o2 + cnt[2]
    k_col = lax.broadcasted_iota(i32, (n, 1), 0).astype(f32)
    s_col = (
        (k_col >= o1).astype(f32)
        + (k_col >= o2).astype(f32)
        + (k_col >= o3).astype(f32)
    )
    start_col = jnp.where(
        s_col == 0.0, 0.0,
        jnp.where(s_col == 1.0, o1, jnp.where(s_col == 2.0, o2, o3)),
    )
    j_int = (k_col - start_col).astype(i32)
    s_int = s_col.astype(i32)
    iota_c = lax.broadcasted_iota(i32, (n, CAP), 1)

    def q_mat(src):
        return ((iota_c == j_int) & (s_int == src)).astype(bf16)

    def cnt_at(src):
        return jnp.where(
            src == 0, cnt[0],
            jnp.where(src == 1, cnt[1], jnp.where(src == 2, cnt[2], cnt[3])),
        )

    acc = lax.dot_general(
        q_mat(me), own_bf, (((1,), (0,)), ((), ())),
        preferred_element_type=f32,
    )
    for k in (1, 3, 2):
        src = (me - k) % N_DEV
        cr = cr_of(cnt_at(src))
        for b, off, sz, active in _chunks(cr):
            recv = pltpu.make_async_remote_copy(
                src_ref=send_ref.at[src, pl.ds(off, sz)],
                dst_ref=stg_ref.at[src, pl.ds(off, sz)],
                send_sem=dsend_sems.at[k, b],
                recv_sem=drecv_sems.at[k, b],
                device_id=(src,),
                device_id_type=pl.DeviceIdType.MESH,
            )
            pl.when(active)(recv.wait_recv)
        acc = acc + lax.dot_general(
            q_mat(src), stg_ref[src],
            (((1,), (0,)), ((), ())), preferred_element_type=f32,
        )
    out_ref[...] = acc

    for rdma in hdr_sends:
        rdma.wait_send()
    for active, rdma in send_chunks:
        pl.when(active)(rdma.wait_send)


def kernel(x, dest):
    n, d = x.shape
    dest_f = dest.astype(jnp.float32)
    dest_row = dest_f.reshape(1, n)
    hdr = dest_f.reshape(HDR, d)

    return pl.pallas_call(
        _a2a_body,
        out_shape=jax.ShapeDtypeStruct((n, d), jnp.float32),
        in_specs=[
            pl.BlockSpec(memory_space=pltpu.VMEM),
            pl.BlockSpec(memory_space=pltpu.VMEM),
            pl.BlockSpec(memory_space=pltpu.VMEM),
        ],
        out_specs=pl.BlockSpec(memory_space=pltpu.VMEM),
        scratch_shapes=[
            pltpu.VMEM((N_DEV, CAP, d), jnp.bfloat16),
            pltpu.VMEM((N_DEV, CAP, d), jnp.bfloat16),
            pltpu.VMEM((N_DEV, HDR, d), jnp.float32),
            pltpu.SemaphoreType.DMA((N_DEV, NBITS)),
            pltpu.SemaphoreType.DMA((N_DEV, NBITS)),
            pltpu.SemaphoreType.DMA((N_DEV,)),
            pltpu.SemaphoreType.DMA((N_DEV,)),
        ],
        compiler_params=pltpu.CompilerParams(collective_id=0),
    )(x, dest_row, hdr)
